# baseline (speedup 1.0000x reference)
import functools

import jax
import jax.numpy as jnp
from jax import lax
from jax.experimental import pallas as pl
from jax.experimental.pallas import tpu as pltpu
from jax.experimental.pallas import tpu_sc as plsc

B2 = 32
B1 = 16
T = 384
E = 256
H = 256
HD = 64
CH = 8
CS = T // CH


def _sc_gather(emb, tok):
    info = plsc.get_sparse_core_info()
    nw = info.num_cores * info.num_subcores
    n = tok.shape[0]
    bpw = n // nw
    mesh = plsc.VectorSubcoreMesh(core_axis_name="c", subcore_axis_name="s")

    @functools.partial(
        pl.kernel,
        mesh=mesh,
        out_type=jax.ShapeDtypeStruct((n, emb.shape[1]), jnp.float32),
        scratch_types=[
            pltpu.VMEM((bpw,), jnp.int32),
            pltpu.VMEM((bpw, emb.shape[1]), jnp.float32),
            pltpu.SemaphoreType.DMA,
        ],
    )
    def k(emb_hbm, tok_hbm, out_hbm, idx_v, rows_v, sem):
        wid = lax.axis_index("s") * info.num_cores + lax.axis_index("c")
        base = wid * bpw
        pltpu.sync_copy(tok_hbm.at[pl.ds(base, bpw)], idx_v)
        pltpu.async_copy(emb_hbm.at[idx_v], rows_v, sem).wait()
        pltpu.sync_copy(rows_v, out_hbm.at[pl.ds(base, bpw)])

    return k(emb, tok)


def _gru_step(h, gi, whh_t, bhh, hid):
    gh = jnp.dot(h.astype(jnp.bfloat16), whh_t,
                 preferred_element_type=jnp.float32) + bhh
    r = jax.nn.sigmoid(gi[:, :hid] + gh[:, :hid])
    z = jax.nn.sigmoid(gi[:, hid:2 * hid] + gh[:, hid:2 * hid])
    n = jnp.tanh(gi[:, 2 * hid:] + r * gh[:, 2 * hid:])
    return (1.0 - z) * n + z * h


def _ed_body(nf_ref, nb_ref, wc_ref, wcb_ref, mf_ref, mb_ref,
             wfih_ref, wfhh_ref, bfih_ref, bfhh_ref,
             wbih_ref, wbhh_ref, bbih_ref, bbhh_ref,
             w1f_ref, w2f_ref, w1b_ref, w2b_ref,
             whh_ref, bih_ref, bhh_ref,
             h2lw_ref, h2lb_ref,
             gmax_ref, e_ref, dmax_ref, y_ref,
             hf_s, hb_s, gif_s, gib_s, gef_s, geb_s, hd_s, g1_s, g2_s):
    c = pl.program_id(0)

    @pl.when(c < CH)
    def _():
        xf = jnp.dot(nf_ref[...].reshape(CS * B2, E).astype(jnp.bfloat16),
                     wc_ref[...],
                     preferred_element_type=jnp.float32) + wcb_ref[...]
        xf = xf.reshape(CS, B2, E) * mf_ref[...][:, :, None]
        xb = jnp.dot(nb_ref[...].reshape(CS * B2, E).astype(jnp.bfloat16),
                     wc_ref[...],
                     preferred_element_type=jnp.float32) + wcb_ref[...]
        xb = xb.reshape(CS, B2, E) * mb_ref[...][:, :, None]
        part = jnp.max(xf, axis=0)
        gif_s[...] = jnp.dot(xf.reshape(CS * B2, E).astype(jnp.bfloat16),
                             wfih_ref[...],
                             preferred_element_type=jnp.float32) + bfih_ref[...]
        gib_s[...] = jnp.dot(xb.reshape(CS * B2, E).astype(jnp.bfloat16),
                             wbih_ref[...],
                             preferred_element_type=jnp.float32) + bbih_ref[...]

        @pl.when(c == 0)
        def _():
            hf_s[...] = jnp.zeros((B2, H), jnp.float32)
            hb_s[...] = jnp.zeros((B2, H), jnp.float32)
            gmax_ref[...] = jnp.full((B2, 2 * H), -jnp.inf, jnp.float32)
            e_ref[...] = part
            hd_s[...] = jnp.zeros((B1, 2 * HD), jnp.float32)
            dmax_ref[...] = jnp.full((B1, 2 * HD), -jnp.inf, jnp.float32)

        @pl.when(c > 0)
        def _():
            e_ref[...] = jnp.maximum(e_ref[...], part)

        wfhh = wfhh_ref[...]
        bfhh = bfhh_ref[...]
        wbhh = wbhh_ref[...]
        bbhh = bbhh_ref[...]

        def step(t, carry):
            hf, hb, gmf, gmb = carry
            gif = gif_s[pl.ds(t * B2, B2), :]
            hf = _gru_step(hf, gif, wfhh, bfhh, H)
            gef_s[pl.ds(c * CS + t, 1)] = hf[None]
            gib = gib_s[pl.ds((CS - 1 - t) * B2, B2), :]
            hb = _gru_step(hb, gib, wbhh, bbhh, H)
            geb_s[pl.ds(c * CS + t, 1)] = hb[None]
            return (hf, hb, jnp.maximum(gmf, hf), jnp.maximum(gmb, hb))

        init = (hf_s[...], hb_s[...], gmax_ref[:, :H], gmax_ref[:, H:])
        hf, hb, gmf, gmb = lax.fori_loop(0, CS, step, init, unroll=8)
        hf_s[...] = hf
        hb_s[...] = hb
        gmax_ref[:, :H] = gmf
        gmax_ref[:, H:] = gmb

    @pl.when(c >= CH)
    def _():
        cc = c - CH
        af = gef_s[pl.ds(cc * CS, CS), 0:B1, :].reshape(CS * B1, H)
        bf = geb_s[pl.ds((CH - 1 - cc) * CS, CS), 0:B1, :].reshape(CS * B1, H)
        ab = gef_s[pl.ds((CH - 1 - cc) * CS, CS), 0:B1, :].reshape(CS * B1, H)
        bb = geb_s[pl.ds(cc * CS, CS), 0:B1, :].reshape(CS * B1, H)
        g1_s[...] = (
            jnp.dot(af.astype(jnp.bfloat16), w1f_ref[...],
                    preferred_element_type=jnp.float32)
            + jnp.dot(bb.astype(jnp.bfloat16), w2b_ref[...],
                      preferred_element_type=jnp.float32)
            + bih_ref[...])
        g2_s[...] = (
            jnp.dot(bf.astype(jnp.bfloat16), w2f_ref[...],
                    preferred_element_type=jnp.float32)
            + jnp.dot(ab.astype(jnp.bfloat16), w1b_ref[...],
                      preferred_element_type=jnp.float32))

        whh = whh_ref[...]
        bhh = bhh_ref[...]
        HC = 2 * HD

        def step(t, carry):
            h, dm = carry
            gi = (g1_s[pl.ds(t * B1, B1), :]
                  + g2_s[pl.ds((CS - 1 - t) * B1, B1), :])
            gh = jnp.dot(h.astype(jnp.bfloat16), whh,
                         preferred_element_type=jnp.float32) + bhh
            r = jax.nn.sigmoid(gi[:, :HC] + gh[:, :HC])
            z = jax.nn.sigmoid(gi[:, HC:2 * HC] + gh[:, HC:2 * HC])
            n = jnp.tanh(gi[:, 2 * HC:] + r * gh[:, 2 * HC:])
            h2 = (1.0 - z) * n + z * h
            return (h2, jnp.maximum(dm, h2))

        init = (hd_s[...], dmax_ref[...])
        h, dm = lax.fori_loop(0, CS, step, init, unroll=8)
        hd_s[...] = h
        dmax_ref[...] = dm

        @pl.when(c == 2 * CH - 1)
        def _():
            g = gmax_ref[...]
            d = jnp.abs(g[:B1, :] - g[B1:, :])
            y_ref[...] = jax.nn.sigmoid(
                jnp.dot(d, h2lw_ref[...], preferred_element_type=jnp.float32)
                + h2lb_ref[...])


def _encode_decode(node_tm, wc_wt, wc_b2, validf_tm,
                   wfih_t, wfhh_t, bfih, bfhh, wbih_t, wbhh_t, bbih, bbhh,
                   w1f_c, w2f_c, w1b_c, w2b_c, whh_cat, bih_cat, bhh_cat,
                   h2lw_t, h2lb):
    ewspec = [
        pl.BlockSpec((E, 3 * H), lambda c: (0, 0)),
        pl.BlockSpec((H, 3 * H), lambda c: (0, 0)),
        pl.BlockSpec((1, 3 * H), lambda c: (0, 0)),
        pl.BlockSpec((1, 3 * H), lambda c: (0, 0)),
    ]
    return pl.pallas_call(
        _ed_body,
        grid=(2 * CH,),
        in_specs=[
            pl.BlockSpec((CS, B2, E), lambda c: (jnp.minimum(c, CH - 1), 0, 0)),
            pl.BlockSpec((CS, B2, E),
                         lambda c: (jnp.maximum(CH - 1 - c, 0), 0, 0)),
            pl.BlockSpec((E, E), lambda c: (0, 0)),
            pl.BlockSpec((1, E), lambda c: (0, 0)),
            pl.BlockSpec((CS, B2), lambda c: (jnp.minimum(c, CH - 1), 0)),
            pl.BlockSpec((CS, B2), lambda c: (jnp.maximum(CH - 1 - c, 0), 0)),
        ] + ewspec + ewspec + [
            pl.BlockSpec((H, 6 * HD), lambda c: (0, 0)),
            pl.BlockSpec((H, 6 * HD), lambda c: (0, 0)),
            pl.BlockSpec((H, 6 * HD), lambda c: (0, 0)),
            pl.BlockSpec((H, 6 * HD), lambda c: (0, 0)),
            pl.BlockSpec((2 * HD, 6 * HD), lambda c: (0, 0)),
            pl.BlockSpec((1, 6 * HD), lambda c: (0, 0)),
            pl.BlockSpec((1, 6 * HD), lambda c: (0, 0)),
            pl.BlockSpec((2 * H, 1), lambda c: (0, 0)),
            pl.BlockSpec((1, 1), lambda c: (0, 0)),
        ],
        out_specs=[
            pl.BlockSpec((B2, 2 * H), lambda c: (0, 0)),
            pl.BlockSpec((B2, E), lambda c: (0, 0)),
            pl.BlockSpec((B1, 2 * HD), lambda c: (0, 0)),
            pl.BlockSpec((B1, 1), lambda c: (0, 0)),
        ],
        out_shape=[
            jax.ShapeDtypeStruct((B2, 2 * H), jnp.float32),
            jax.ShapeDtypeStruct((B2, E), jnp.float32),
            jax.ShapeDtypeStruct((B1, 2 * HD), jnp.float32),
            jax.ShapeDtypeStruct((B1, 1), jnp.float32),
        ],
        scratch_shapes=[
            pltpu.VMEM((B2, H), jnp.float32),
            pltpu.VMEM((B2, H), jnp.float32),
            pltpu.VMEM((CS * B2, 3 * H), jnp.float32),
            pltpu.VMEM((CS * B2, 3 * H), jnp.float32),
            pltpu.VMEM((T, B2, H), jnp.float32),
            pltpu.VMEM((T, B2, H), jnp.float32),
            pltpu.VMEM((B1, 2 * HD), jnp.float32),
            pltpu.VMEM((CS * B1, 6 * HD), jnp.float32),
            pltpu.VMEM((CS * B1, 6 * HD), jnp.float32),
        ],
    )(node_tm, node_tm, wc_wt, wc_b2, validf_tm, validf_tm,
      wfih_t, wfhh_t, bfih, bfhh, wbih_t, wbhh_t, bbih, bbhh,
      w1f_c, w2f_c, w1b_c, w2b_c, whh_cat, bih_cat, bhh_cat,
      h2lw_t, h2lb)


def kernel(tokens1, cu_seqlens1, tokens2, cu_seqlens2, emb, wc_w, wc_b,
           ge_wih_f, ge_whh_f, ge_bih_f, ge_bhh_f,
           ge_wih_b, ge_whh_b, ge_bih_b, ge_bhh_b,
           gd_wih_f, gd_whh_f, gd_bih_f, gd_bhh_f,
           gd_wih_b, gd_whh_b, gd_bih_b, gd_bhh_b,
           h2l_w, h2l_b):
    n1 = tokens1.shape[0]
    tokens = jnp.concatenate([tokens1, tokens2]).astype(jnp.int32)
    starts = jnp.concatenate([cu_seqlens1[:-1], cu_seqlens2[:-1] + n1])
    lens = jnp.concatenate([cu_seqlens1[1:] - cu_seqlens1[:-1],
                            cu_seqlens2[1:] - cu_seqlens2[:-1]])
    pad = T - lens
    j = jnp.arange(T, dtype=jnp.int32)
    idx = starts[:, None] + j[None, :] - pad[:, None]
    valid = j[None, :] >= pad[:, None]
    idxc = jnp.clip(idx, 0, tokens.shape[0] - 1).astype(jnp.int32)
    tok_padded = jnp.take(tokens, idxc.T.reshape(-1), axis=0)
    node = _sc_gather(emb, tok_padded)

    validf_tm = valid.T.astype(jnp.float32)
    bf16 = jnp.bfloat16

    def _ilv(wf, wb):
        parts = []
        for g in range(3):
            parts.append(wf[:, g * HD:(g + 1) * HD])
            parts.append(wb[:, g * HD:(g + 1) * HD])
        return jnp.concatenate(parts, axis=1)

    zih = jnp.zeros((H, 3 * HD), jnp.float32)
    zhh = jnp.zeros((HD, 3 * HD), jnp.float32)
    whh_cat = jnp.concatenate([
        _ilv(gd_whh_f.T, zhh), _ilv(zhh, gd_whh_b.T)], axis=0).astype(bf16)
    bih_cat = _ilv(gd_bih_f.reshape(1, -1), gd_bih_b.reshape(1, -1))
    bhh_cat = _ilv(gd_bhh_f.reshape(1, -1), gd_bhh_b.reshape(1, -1))

    gmax, e, dmax, y = _encode_decode(
        node.reshape(T, B2, E), wc_w.T.astype(bf16), wc_b.reshape(1, -1),
        validf_tm,
        ge_wih_f.T.astype(bf16), ge_whh_f.T.astype(bf16),
        ge_bih_f.reshape(1, -1), ge_bhh_f.reshape(1, -1),
        ge_wih_b.T.astype(bf16), ge_whh_b.T.astype(bf16),
        ge_bih_b.reshape(1, -1), ge_bhh_b.reshape(1, -1),
        _ilv(gd_wih_f[:, :H].T, zih).astype(bf16),
        _ilv(gd_wih_f[:, H:].T, zih).astype(bf16),
        _ilv(zih, gd_wih_b[:, :H].T).astype(bf16),
        _ilv(zih, gd_wih_b[:, H:].T).astype(bf16),
        whh_cat, bih_cat, bhh_cat,
        h2l_w.T, h2l_b.reshape(1, 1))

    return (e[:B1], gmax[:B1], dmax, y)


# speedup vs baseline: 1.0145x; 1.0145x over previous
import functools

import jax
import jax.numpy as jnp
from jax import lax
from jax.experimental import pallas as pl
from jax.experimental.pallas import tpu as pltpu
from jax.experimental.pallas import tpu_sc as plsc

B2 = 32
B1 = 16
T = 384
E = 256
H = 256
HD = 64
CH = 8
CS = T // CH


def _sc_gather(emb, tok):
    info = plsc.get_sparse_core_info()
    nw = info.num_cores * info.num_subcores
    n = tok.shape[0]
    bpw = n // nw
    mesh = plsc.VectorSubcoreMesh(core_axis_name="c", subcore_axis_name="s")

    @functools.partial(
        pl.kernel,
        mesh=mesh,
        out_type=jax.ShapeDtypeStruct((n, emb.shape[1]), jnp.float32),
        scratch_types=[
            pltpu.VMEM((bpw,), jnp.int32),
            pltpu.VMEM((bpw, emb.shape[1]), jnp.float32),
            pltpu.SemaphoreType.DMA,
        ],
    )
    def k(emb_hbm, tok_hbm, out_hbm, idx_v, rows_v, sem):
        wid = lax.axis_index("s") * info.num_cores + lax.axis_index("c")
        base = wid * bpw
        pltpu.sync_copy(tok_hbm.at[pl.ds(base, bpw)], idx_v)
        pltpu.async_copy(emb_hbm.at[idx_v], rows_v, sem).wait()
        pltpu.sync_copy(rows_v, out_hbm.at[pl.ds(base, bpw)])

    return k(emb, tok)


def _gru_step(h, gi, whh_t, bhh, hid):
    gh = jnp.dot(h.astype(jnp.bfloat16), whh_t,
                 preferred_element_type=jnp.float32) + bhh
    r = jax.nn.sigmoid(gi[:, :hid] + gh[:, :hid])
    z = jax.nn.sigmoid(gi[:, hid:2 * hid] + gh[:, hid:2 * hid])
    n = jnp.tanh(gi[:, 2 * hid:] + r * gh[:, 2 * hid:])
    return (1.0 - z) * n + z * h


def _ed_body(nf_ref, nb_ref, wc_ref, wcb_ref, mf_ref, mb_ref,
             wfih_ref, wfhh_ref, bfih_ref, bfhh_ref,
             wbih_ref, wbhh_ref, bbih_ref, bbhh_ref,
             w1f_ref, w2f_ref, w1b_ref, w2b_ref,
             whh_ref, bih_ref, bhh_ref,
             h2lw_ref, h2lb_ref,
             gmax_ref, e_ref, dmax_ref, y_ref,
             hf_s, hb_s, gif_s, gib_s, gef_s, geb_s, hd_s, g1_s, g2_s):
    c = pl.program_id(0)

    @pl.when(c < CH)
    def _():
        xf = jnp.dot(nf_ref[...].reshape(CS * B2, E).astype(jnp.bfloat16),
                     wc_ref[...],
                     preferred_element_type=jnp.float32) + wcb_ref[...]
        xf = xf.reshape(CS, B2, E) * mf_ref[...][:, :, None]
        xb = jnp.dot(nb_ref[...].reshape(CS * B2, E).astype(jnp.bfloat16),
                     wc_ref[...],
                     preferred_element_type=jnp.float32) + wcb_ref[...]
        xb = xb.reshape(CS, B2, E) * mb_ref[...][:, :, None]
        part = jnp.max(xf, axis=0)
        gif_s[...] = jnp.dot(xf.reshape(CS * B2, E).astype(jnp.bfloat16),
                             wfih_ref[...],
                             preferred_element_type=jnp.float32) + bfih_ref[...]
        gib_s[...] = jnp.dot(xb.reshape(CS * B2, E).astype(jnp.bfloat16),
                             wbih_ref[...],
                             preferred_element_type=jnp.float32) + bbih_ref[...]

        @pl.when(c == 0)
        def _():
            hf_s[...] = jnp.zeros((B2, H), jnp.float32)
            hb_s[...] = jnp.zeros((B2, H), jnp.float32)
            gmax_ref[...] = jnp.full((B2, 2 * H), -jnp.inf, jnp.float32)
            e_ref[...] = part
            hd_s[...] = jnp.zeros((B1, 2 * HD), jnp.float32)
            dmax_ref[...] = jnp.full((B1, 2 * HD), -jnp.inf, jnp.float32)

        @pl.when(c > 0)
        def _():
            e_ref[...] = jnp.maximum(e_ref[...], part)

        wfhh = wfhh_ref[...]
        bfhh = bfhh_ref[...]
        wbhh = wbhh_ref[...]
        bbhh = bbhh_ref[...]

        def step(t, carry):
            hf, hb, gmf, gmb = carry
            gif = gif_s[pl.ds(t * B2, B2), :]
            hf = _gru_step(hf, gif, wfhh, bfhh, H)
            gef_s[pl.ds(c * CS + t, 1)] = hf[None]
            gib = gib_s[pl.ds((CS - 1 - t) * B2, B2), :]
            hb = _gru_step(hb, gib, wbhh, bbhh, H)
            geb_s[pl.ds(c * CS + t, 1)] = hb[None]
            return (hf, hb, jnp.maximum(gmf, hf), jnp.maximum(gmb, hb))

        init = (hf_s[...], hb_s[...], gmax_ref[:, :H], gmax_ref[:, H:])
        hf, hb, gmf, gmb = lax.fori_loop(0, CS, step, init, unroll=16)
        hf_s[...] = hf
        hb_s[...] = hb
        gmax_ref[:, :H] = gmf
        gmax_ref[:, H:] = gmb

    @pl.when(c >= CH)
    def _():
        cc = c - CH
        af = gef_s[pl.ds(cc * CS, CS), 0:B1, :].reshape(CS * B1, H)
        bf = geb_s[pl.ds((CH - 1 - cc) * CS, CS), 0:B1, :].reshape(CS * B1, H)
        ab = gef_s[pl.ds((CH - 1 - cc) * CS, CS), 0:B1, :].reshape(CS * B1, H)
        bb = geb_s[pl.ds(cc * CS, CS), 0:B1, :].reshape(CS * B1, H)
        g1_s[...] = (
            jnp.dot(af.astype(jnp.bfloat16), w1f_ref[...],
                    preferred_element_type=jnp.float32)
            + jnp.dot(bb.astype(jnp.bfloat16), w2b_ref[...],
                      preferred_element_type=jnp.float32)
            + bih_ref[...])
        g2_s[...] = (
            jnp.dot(bf.astype(jnp.bfloat16), w2f_ref[...],
                    preferred_element_type=jnp.float32)
            + jnp.dot(ab.astype(jnp.bfloat16), w1b_ref[...],
                      preferred_element_type=jnp.float32))

        whh = whh_ref[...]
        bhh = bhh_ref[...]
        HC = 2 * HD

        def step(t, carry):
            h, dm = carry
            gi = (g1_s[pl.ds(t * B1, B1), :]
                  + g2_s[pl.ds((CS - 1 - t) * B1, B1), :])
            gh = jnp.dot(h.astype(jnp.bfloat16), whh,
                         preferred_element_type=jnp.float32) + bhh
            r = jax.nn.sigmoid(gi[:, :HC] + gh[:, :HC])
            z = jax.nn.sigmoid(gi[:, HC:2 * HC] + gh[:, HC:2 * HC])
            n = jnp.tanh(gi[:, 2 * HC:] + r * gh[:, 2 * HC:])
            h2 = (1.0 - z) * n + z * h
            return (h2, jnp.maximum(dm, h2))

        init = (hd_s[...], dmax_ref[...])
        h, dm = lax.fori_loop(0, CS, step, init, unroll=16)
        hd_s[...] = h
        dmax_ref[...] = dm

        @pl.when(c == 2 * CH - 1)
        def _():
            g = gmax_ref[...]
            d = jnp.abs(g[:B1, :] - g[B1:, :])
            y_ref[...] = jax.nn.sigmoid(
                jnp.dot(d, h2lw_ref[...], preferred_element_type=jnp.float32)
                + h2lb_ref[...])


def _encode_decode(node_tm, wc_wt, wc_b2, validf_tm,
                   wfih_t, wfhh_t, bfih, bfhh, wbih_t, wbhh_t, bbih, bbhh,
                   w1f_c, w2f_c, w1b_c, w2b_c, whh_cat, bih_cat, bhh_cat,
                   h2lw_t, h2lb):
    ewspec = [
        pl.BlockSpec((E, 3 * H), lambda c: (0, 0)),
        pl.BlockSpec((H, 3 * H), lambda c: (0, 0)),
        pl.BlockSpec((1, 3 * H), lambda c: (0, 0)),
        pl.BlockSpec((1, 3 * H), lambda c: (0, 0)),
    ]
    return pl.pallas_call(
        _ed_body,
        grid=(2 * CH,),
        in_specs=[
            pl.BlockSpec((CS, B2, E), lambda c: (jnp.minimum(c, CH - 1), 0, 0)),
            pl.BlockSpec((CS, B2, E),
                         lambda c: (jnp.maximum(CH - 1 - c, 0), 0, 0)),
            pl.BlockSpec((E, E), lambda c: (0, 0)),
            pl.BlockSpec((1, E), lambda c: (0, 0)),
            pl.BlockSpec((CS, B2), lambda c: (jnp.minimum(c, CH - 1), 0)),
            pl.BlockSpec((CS, B2), lambda c: (jnp.maximum(CH - 1 - c, 0), 0)),
        ] + ewspec + ewspec + [
            pl.BlockSpec((H, 6 * HD), lambda c: (0, 0)),
            pl.BlockSpec((H, 6 * HD), lambda c: (0, 0)),
            pl.BlockSpec((H, 6 * HD), lambda c: (0, 0)),
            pl.BlockSpec((H, 6 * HD), lambda c: (0, 0)),
            pl.BlockSpec((2 * HD, 6 * HD), lambda c: (0, 0)),
            pl.BlockSpec((1, 6 * HD), lambda c: (0, 0)),
            pl.BlockSpec((1, 6 * HD), lambda c: (0, 0)),
            pl.BlockSpec((2 * H, 1), lambda c: (0, 0)),
            pl.BlockSpec((1, 1), lambda c: (0, 0)),
        ],
        out_specs=[
            pl.BlockSpec((B2, 2 * H), lambda c: (0, 0)),
            pl.BlockSpec((B2, E), lambda c: (0, 0)),
            pl.BlockSpec((B1, 2 * HD), lambda c: (0, 0)),
            pl.BlockSpec((B1, 1), lambda c: (0, 0)),
        ],
        out_shape=[
            jax.ShapeDtypeStruct((B2, 2 * H), jnp.float32),
            jax.ShapeDtypeStruct((B2, E), jnp.float32),
            jax.ShapeDtypeStruct((B1, 2 * HD), jnp.float32),
            jax.ShapeDtypeStruct((B1, 1), jnp.float32),
        ],
        scratch_shapes=[
            pltpu.VMEM((B2, H), jnp.float32),
            pltpu.VMEM((B2, H), jnp.float32),
            pltpu.VMEM((CS * B2, 3 * H), jnp.float32),
            pltpu.VMEM((CS * B2, 3 * H), jnp.float32),
            pltpu.VMEM((T, B2, H), jnp.float32),
            pltpu.VMEM((T, B2, H), jnp.float32),
            pltpu.VMEM((B1, 2 * HD), jnp.float32),
            pltpu.VMEM((CS * B1, 6 * HD), jnp.float32),
            pltpu.VMEM((CS * B1, 6 * HD), jnp.float32),
        ],
    )(node_tm, node_tm, wc_wt, wc_b2, validf_tm, validf_tm,
      wfih_t, wfhh_t, bfih, bfhh, wbih_t, wbhh_t, bbih, bbhh,
      w1f_c, w2f_c, w1b_c, w2b_c, whh_cat, bih_cat, bhh_cat,
      h2lw_t, h2lb)


def kernel(tokens1, cu_seqlens1, tokens2, cu_seqlens2, emb, wc_w, wc_b,
           ge_wih_f, ge_whh_f, ge_bih_f, ge_bhh_f,
           ge_wih_b, ge_whh_b, ge_bih_b, ge_bhh_b,
           gd_wih_f, gd_whh_f, gd_bih_f, gd_bhh_f,
           gd_wih_b, gd_whh_b, gd_bih_b, gd_bhh_b,
           h2l_w, h2l_b):
    n1 = tokens1.shape[0]
    tokens = jnp.concatenate([tokens1, tokens2]).astype(jnp.int32)
    starts = jnp.concatenate([cu_seqlens1[:-1], cu_seqlens2[:-1] + n1])
    lens = jnp.concatenate([cu_seqlens1[1:] - cu_seqlens1[:-1],
                            cu_seqlens2[1:] - cu_seqlens2[:-1]])
    pad = T - lens
    j = jnp.arange(T, dtype=jnp.int32)
    idx = starts[:, None] + j[None, :] - pad[:, None]
    valid = j[None, :] >= pad[:, None]
    idxc = jnp.clip(idx, 0, tokens.shape[0] - 1).astype(jnp.int32)
    tok_padded = jnp.take(tokens, idxc.T.reshape(-1), axis=0)
    node = _sc_gather(emb, tok_padded)

    validf_tm = valid.T.astype(jnp.float32)
    bf16 = jnp.bfloat16

    def _ilv(wf, wb):
        parts = []
        for g in range(3):
            parts.append(wf[:, g * HD:(g + 1) * HD])
            parts.append(wb[:, g * HD:(g + 1) * HD])
        return jnp.concatenate(parts, axis=1)

    zih = jnp.zeros((H, 3 * HD), jnp.float32)
    zhh = jnp.zeros((HD, 3 * HD), jnp.float32)
    whh_cat = jnp.concatenate([
        _ilv(gd_whh_f.T, zhh), _ilv(zhh, gd_whh_b.T)], axis=0).astype(bf16)
    bih_cat = _ilv(gd_bih_f.reshape(1, -1), gd_bih_b.reshape(1, -1))
    bhh_cat = _ilv(gd_bhh_f.reshape(1, -1), gd_bhh_b.reshape(1, -1))

    gmax, e, dmax, y = _encode_decode(
        node.reshape(T, B2, E), wc_w.T.astype(bf16), wc_b.reshape(1, -1),
        validf_tm,
        ge_wih_f.T.astype(bf16), ge_whh_f.T.astype(bf16),
        ge_bih_f.reshape(1, -1), ge_bhh_f.reshape(1, -1),
        ge_wih_b.T.astype(bf16), ge_whh_b.T.astype(bf16),
        ge_bih_b.reshape(1, -1), ge_bhh_b.reshape(1, -1),
        _ilv(gd_wih_f[:, :H].T, zih).astype(bf16),
        _ilv(gd_wih_f[:, H:].T, zih).astype(bf16),
        _ilv(zih, gd_wih_b[:, :H].T).astype(bf16),
        _ilv(zih, gd_wih_b[:, H:].T).astype(bf16),
        whh_cat, bih_cat, bhh_cat,
        h2l_w.T, h2l_b.reshape(1, 1))

    return (e[:B1], gmax[:B1], dmax, y)


# speedup vs baseline: 1.0185x; 1.0040x over previous
import functools

import jax
import jax.numpy as jnp
from jax import lax
from jax.experimental import pallas as pl
from jax.experimental.pallas import tpu as pltpu
from jax.experimental.pallas import tpu_sc as plsc

B2 = 32
B1 = 16
T = 384
E = 256
H = 256
HD = 64
CH = 8
CS = T // CH


def _sc_gather(emb, tok):
    info = plsc.get_sparse_core_info()
    nw = info.num_cores * info.num_subcores
    n = tok.shape[0]
    bpw = n // nw
    mesh = plsc.VectorSubcoreMesh(core_axis_name="c", subcore_axis_name="s")

    @functools.partial(
        pl.kernel,
        mesh=mesh,
        out_type=jax.ShapeDtypeStruct((n, emb.shape[1]), jnp.float32),
        scratch_types=[
            pltpu.VMEM((bpw,), jnp.int32),
            pltpu.VMEM((bpw, emb.shape[1]), jnp.float32),
            pltpu.SemaphoreType.DMA,
        ],
    )
    def k(emb_hbm, tok_hbm, out_hbm, idx_v, rows_v, sem):
        wid = lax.axis_index("s") * info.num_cores + lax.axis_index("c")
        base = wid * bpw
        pltpu.sync_copy(tok_hbm.at[pl.ds(base, bpw)], idx_v)
        pltpu.async_copy(emb_hbm.at[idx_v], rows_v, sem).wait()
        pltpu.sync_copy(rows_v, out_hbm.at[pl.ds(base, bpw)])

    return k(emb, tok)


def _gru_step(h, gi, whh_t, bhh, hid):
    gh = jnp.dot(h.astype(jnp.bfloat16), whh_t,
                 preferred_element_type=jnp.float32) + bhh
    r = jax.nn.sigmoid(gi[:, :hid] + gh[:, :hid])
    z = jax.nn.sigmoid(gi[:, hid:2 * hid] + gh[:, hid:2 * hid])
    n = jnp.tanh(gi[:, 2 * hid:] + r * gh[:, 2 * hid:])
    return (1.0 - z) * n + z * h


def _ed_body(nf_ref, nb_ref, wc_ref, wcb_ref, mf_ref, mb_ref,
             wfih_ref, wfhh_ref, bfih_ref, bfhh_ref,
             wbih_ref, wbhh_ref, bbih_ref, bbhh_ref,
             w1f_ref, w2f_ref, w1b_ref, w2b_ref,
             whh_ref, bih_ref, bhh_ref,
             h2lw_ref, h2lb_ref,
             gmax_ref, e_ref, dmax_ref, y_ref,
             hf_s, hb_s, gif_s, gib_s, gef_s, geb_s, hd_s, g1_s, g2_s):
    c = pl.program_id(0)

    @pl.when(c < CH)
    def _():
        xf = jnp.dot(nf_ref[...].reshape(CS * B2, E).astype(jnp.bfloat16),
                     wc_ref[...],
                     preferred_element_type=jnp.float32) + wcb_ref[...]
        xf = xf.reshape(CS, B2, E) * mf_ref[...][:, :, None]
        xb = jnp.dot(nb_ref[...].reshape(CS * B2, E).astype(jnp.bfloat16),
                     wc_ref[...],
                     preferred_element_type=jnp.float32) + wcb_ref[...]
        xb = xb.reshape(CS, B2, E) * mb_ref[...][:, :, None]
        part = jnp.max(xf, axis=0)
        gif_s[...] = jnp.dot(xf.reshape(CS * B2, E).astype(jnp.bfloat16),
                             wfih_ref[...],
                             preferred_element_type=jnp.float32) + bfih_ref[...]
        gib_s[...] = jnp.dot(xb.reshape(CS * B2, E).astype(jnp.bfloat16),
                             wbih_ref[...],
                             preferred_element_type=jnp.float32) + bbih_ref[...]

        @pl.when(c == 0)
        def _():
            hf_s[...] = jnp.zeros((B2, H), jnp.float32)
            hb_s[...] = jnp.zeros((B2, H), jnp.float32)
            gmax_ref[...] = jnp.full((B2, 2 * H), -jnp.inf, jnp.float32)
            e_ref[...] = part
            hd_s[...] = jnp.zeros((B1, 2 * HD), jnp.float32)
            dmax_ref[...] = jnp.full((B1, 2 * HD), -jnp.inf, jnp.float32)

        @pl.when(c > 0)
        def _():
            e_ref[...] = jnp.maximum(e_ref[...], part)

        wfhh = wfhh_ref[...]
        bfhh = bfhh_ref[...]
        wbhh = wbhh_ref[...]
        bbhh = bbhh_ref[...]

        def step(t, carry):
            hf, hb, gmf, gmb = carry
            gif = gif_s[pl.ds(t * B2, B2), :]
            hf = _gru_step(hf, gif, wfhh, bfhh, H)
            gef_s[pl.ds(c * CS + t, 1)] = hf[None]
            gib = gib_s[pl.ds((CS - 1 - t) * B2, B2), :]
            hb = _gru_step(hb, gib, wbhh, bbhh, H)
            geb_s[pl.ds(c * CS + t, 1)] = hb[None]
            return (hf, hb, jnp.maximum(gmf, hf), jnp.maximum(gmb, hb))

        init = (hf_s[...], hb_s[...], gmax_ref[:, :H], gmax_ref[:, H:])
        hf, hb, gmf, gmb = lax.fori_loop(0, CS, step, init, unroll=24)
        hf_s[...] = hf
        hb_s[...] = hb
        gmax_ref[:, :H] = gmf
        gmax_ref[:, H:] = gmb

    @pl.when(c >= CH)
    def _():
        cc = c - CH
        af = gef_s[pl.ds(cc * CS, CS), 0:B1, :].reshape(CS * B1, H)
        bf = geb_s[pl.ds((CH - 1 - cc) * CS, CS), 0:B1, :].reshape(CS * B1, H)
        ab = gef_s[pl.ds((CH - 1 - cc) * CS, CS), 0:B1, :].reshape(CS * B1, H)
        bb = geb_s[pl.ds(cc * CS, CS), 0:B1, :].reshape(CS * B1, H)
        g1_s[...] = (
            jnp.dot(af.astype(jnp.bfloat16), w1f_ref[...],
                    preferred_element_type=jnp.float32)
            + jnp.dot(bb.astype(jnp.bfloat16), w2b_ref[...],
                      preferred_element_type=jnp.float32)
            + bih_ref[...])
        g2_s[...] = (
            jnp.dot(bf.astype(jnp.bfloat16), w2f_ref[...],
                    preferred_element_type=jnp.float32)
            + jnp.dot(ab.astype(jnp.bfloat16), w1b_ref[...],
                      preferred_element_type=jnp.float32))

        whh = whh_ref[...]
        bhh = bhh_ref[...]
        HC = 2 * HD

        def step(t, carry):
            h, dm = carry
            gi = (g1_s[pl.ds(t * B1, B1), :]
                  + g2_s[pl.ds((CS - 1 - t) * B1, B1), :])
            gh = jnp.dot(h.astype(jnp.bfloat16), whh,
                         preferred_element_type=jnp.float32) + bhh
            r = jax.nn.sigmoid(gi[:, :HC] + gh[:, :HC])
            z = jax.nn.sigmoid(gi[:, HC:2 * HC] + gh[:, HC:2 * HC])
            n = jnp.tanh(gi[:, 2 * HC:] + r * gh[:, 2 * HC:])
            h2 = (1.0 - z) * n + z * h
            return (h2, jnp.maximum(dm, h2))

        init = (hd_s[...], dmax_ref[...])
        h, dm = lax.fori_loop(0, CS, step, init, unroll=24)
        hd_s[...] = h
        dmax_ref[...] = dm

        @pl.when(c == 2 * CH - 1)
        def _():
            g = gmax_ref[...]
            d = jnp.abs(g[:B1, :] - g[B1:, :])
            y_ref[...] = jax.nn.sigmoid(
                jnp.dot(d, h2lw_ref[...], preferred_element_type=jnp.float32)
                + h2lb_ref[...])


def _encode_decode(node_tm, wc_wt, wc_b2, validf_tm,
                   wfih_t, wfhh_t, bfih, bfhh, wbih_t, wbhh_t, bbih, bbhh,
                   w1f_c, w2f_c, w1b_c, w2b_c, whh_cat, bih_cat, bhh_cat,
                   h2lw_t, h2lb):
    ewspec = [
        pl.BlockSpec((E, 3 * H), lambda c: (0, 0)),
        pl.BlockSpec((H, 3 * H), lambda c: (0, 0)),
        pl.BlockSpec((1, 3 * H), lambda c: (0, 0)),
        pl.BlockSpec((1, 3 * H), lambda c: (0, 0)),
    ]
    return pl.pallas_call(
        _ed_body,
        grid=(2 * CH,),
        in_specs=[
            pl.BlockSpec((CS, B2, E), lambda c: (jnp.minimum(c, CH - 1), 0, 0)),
            pl.BlockSpec((CS, B2, E),
                         lambda c: (jnp.maximum(CH - 1 - c, 0), 0, 0)),
            pl.BlockSpec((E, E), lambda c: (0, 0)),
            pl.BlockSpec((1, E), lambda c: (0, 0)),
            pl.BlockSpec((CS, B2), lambda c: (jnp.minimum(c, CH - 1), 0)),
            pl.BlockSpec((CS, B2), lambda c: (jnp.maximum(CH - 1 - c, 0), 0)),
        ] + ewspec + ewspec + [
            pl.BlockSpec((H, 6 * HD), lambda c: (0, 0)),
            pl.BlockSpec((H, 6 * HD), lambda c: (0, 0)),
            pl.BlockSpec((H, 6 * HD), lambda c: (0, 0)),
            pl.BlockSpec((H, 6 * HD), lambda c: (0, 0)),
            pl.BlockSpec((2 * HD, 6 * HD), lambda c: (0, 0)),
            pl.BlockSpec((1, 6 * HD), lambda c: (0, 0)),
            pl.BlockSpec((1, 6 * HD), lambda c: (0, 0)),
            pl.BlockSpec((2 * H, 1), lambda c: (0, 0)),
            pl.BlockSpec((1, 1), lambda c: (0, 0)),
        ],
        out_specs=[
            pl.BlockSpec((B2, 2 * H), lambda c: (0, 0)),
            pl.BlockSpec((B2, E), lambda c: (0, 0)),
            pl.BlockSpec((B1, 2 * HD), lambda c: (0, 0)),
            pl.BlockSpec((B1, 1), lambda c: (0, 0)),
        ],
        out_shape=[
            jax.ShapeDtypeStruct((B2, 2 * H), jnp.float32),
            jax.ShapeDtypeStruct((B2, E), jnp.float32),
            jax.ShapeDtypeStruct((B1, 2 * HD), jnp.float32),
            jax.ShapeDtypeStruct((B1, 1), jnp.float32),
        ],
        scratch_shapes=[
            pltpu.VMEM((B2, H), jnp.float32),
            pltpu.VMEM((B2, H), jnp.float32),
            pltpu.VMEM((CS * B2, 3 * H), jnp.float32),
            pltpu.VMEM((CS * B2, 3 * H), jnp.float32),
            pltpu.VMEM((T, B2, H), jnp.float32),
            pltpu.VMEM((T, B2, H), jnp.float32),
            pltpu.VMEM((B1, 2 * HD), jnp.float32),
            pltpu.VMEM((CS * B1, 6 * HD), jnp.float32),
            pltpu.VMEM((CS * B1, 6 * HD), jnp.float32),
        ],
    )(node_tm, node_tm, wc_wt, wc_b2, validf_tm, validf_tm,
      wfih_t, wfhh_t, bfih, bfhh, wbih_t, wbhh_t, bbih, bbhh,
      w1f_c, w2f_c, w1b_c, w2b_c, whh_cat, bih_cat, bhh_cat,
      h2lw_t, h2lb)


def kernel(tokens1, cu_seqlens1, tokens2, cu_seqlens2, emb, wc_w, wc_b,
           ge_wih_f, ge_whh_f, ge_bih_f, ge_bhh_f,
           ge_wih_b, ge_whh_b, ge_bih_b, ge_bhh_b,
           gd_wih_f, gd_whh_f, gd_bih_f, gd_bhh_f,
           gd_wih_b, gd_whh_b, gd_bih_b, gd_bhh_b,
           h2l_w, h2l_b):
    n1 = tokens1.shape[0]
    tokens = jnp.concatenate([tokens1, tokens2]).astype(jnp.int32)
    starts = jnp.concatenate([cu_seqlens1[:-1], cu_seqlens2[:-1] + n1])
    lens = jnp.concatenate([cu_seqlens1[1:] - cu_seqlens1[:-1],
                            cu_seqlens2[1:] - cu_seqlens2[:-1]])
    pad = T - lens
    j = jnp.arange(T, dtype=jnp.int32)
    idx = starts[:, None] + j[None, :] - pad[:, None]
    valid = j[None, :] >= pad[:, None]
    idxc = jnp.clip(idx, 0, tokens.shape[0] - 1).astype(jnp.int32)
    tok_padded = jnp.take(tokens, idxc.T.reshape(-1), axis=0)
    node = _sc_gather(emb, tok_padded)

    validf_tm = valid.T.astype(jnp.float32)
    bf16 = jnp.bfloat16

    def _ilv(wf, wb):
        parts = []
        for g in range(3):
            parts.append(wf[:, g * HD:(g + 1) * HD])
            parts.append(wb[:, g * HD:(g + 1) * HD])
        return jnp.concatenate(parts, axis=1)

    zih = jnp.zeros((H, 3 * HD), jnp.float32)
    zhh = jnp.zeros((HD, 3 * HD), jnp.float32)
    whh_cat = jnp.concatenate([
        _ilv(gd_whh_f.T, zhh), _ilv(zhh, gd_whh_b.T)], axis=0).astype(bf16)
    bih_cat = _ilv(gd_bih_f.reshape(1, -1), gd_bih_b.reshape(1, -1))
    bhh_cat = _ilv(gd_bhh_f.reshape(1, -1), gd_bhh_b.reshape(1, -1))

    gmax, e, dmax, y = _encode_decode(
        node.reshape(T, B2, E), wc_w.T.astype(bf16), wc_b.reshape(1, -1),
        validf_tm,
        ge_wih_f.T.astype(bf16), ge_whh_f.T.astype(bf16),
        ge_bih_f.reshape(1, -1), ge_bhh_f.reshape(1, -1),
        ge_wih_b.T.astype(bf16), ge_whh_b.T.astype(bf16),
        ge_bih_b.reshape(1, -1), ge_bhh_b.reshape(1, -1),
        _ilv(gd_wih_f[:, :H].T, zih).astype(bf16),
        _ilv(gd_wih_f[:, H:].T, zih).astype(bf16),
        _ilv(zih, gd_wih_b[:, :H].T).astype(bf16),
        _ilv(zih, gd_wih_b[:, H:].T).astype(bf16),
        whh_cat, bih_cat, bhh_cat,
        h2l_w.T, h2l_b.reshape(1, 1))

    return (e[:B1], gmax[:B1], dmax, y)


# speedup vs baseline: 1.0200x; 1.0015x over previous
import functools

import jax
import jax.numpy as jnp
from jax import lax
from jax.experimental import pallas as pl
from jax.experimental.pallas import tpu as pltpu
from jax.experimental.pallas import tpu_sc as plsc

B2 = 32
B1 = 16
T = 384
E = 256
H = 256
HD = 64
CH = 8
CS = T // CH


def _sc_gather(emb, tok):
    info = plsc.get_sparse_core_info()
    nw = info.num_cores * info.num_subcores
    n = tok.shape[0]
    bpw = n // nw
    mesh = plsc.VectorSubcoreMesh(core_axis_name="c", subcore_axis_name="s")

    @functools.partial(
        pl.kernel,
        mesh=mesh,
        out_type=jax.ShapeDtypeStruct((n, emb.shape[1]), jnp.float32),
        scratch_types=[
            pltpu.VMEM((bpw,), jnp.int32),
            pltpu.VMEM((bpw, emb.shape[1]), jnp.float32),
            pltpu.SemaphoreType.DMA,
        ],
    )
    def k(emb_hbm, tok_hbm, out_hbm, idx_v, rows_v, sem):
        wid = lax.axis_index("s") * info.num_cores + lax.axis_index("c")
        base = wid * bpw
        pltpu.sync_copy(tok_hbm.at[pl.ds(base, bpw)], idx_v)
        pltpu.async_copy(emb_hbm.at[idx_v], rows_v, sem).wait()
        pltpu.sync_copy(rows_v, out_hbm.at[pl.ds(base, bpw)])

    return k(emb, tok)


def _gru_step(h, gi, whh_t, bhh, hid):
    gh = jnp.dot(h.astype(jnp.bfloat16), whh_t,
                 preferred_element_type=jnp.float32) + bhh
    r = jax.nn.sigmoid(gi[:, :hid] + gh[:, :hid])
    z = jax.nn.sigmoid(gi[:, hid:2 * hid] + gh[:, hid:2 * hid])
    n = jnp.tanh(gi[:, 2 * hid:] + r * gh[:, 2 * hid:])
    return (1.0 - z) * n + z * h


def _ed_body(nf_ref, nb_ref, wc_ref, wcb_ref, mf_ref, mb_ref,
             wfih_ref, wfhh_ref, bfih_ref, bfhh_ref,
             wbih_ref, wbhh_ref, bbih_ref, bbhh_ref,
             w1f_ref, w2f_ref, w1b_ref, w2b_ref,
             whh_ref, bih_ref, bhh_ref,
             h2lw_ref, h2lb_ref,
             gmax_ref, e_ref, dmax_ref, y_ref,
             hf_s, hb_s, gif_s, gib_s, gef_s, geb_s, hd_s, g1_s, g2_s, x_s):
    c = pl.program_id(0)

    @pl.when(c < CH)
    def _():
        @pl.when(c < CH // 2)
        def _():
            pf = jnp.dot(nf_ref[...].reshape(CS * B2, E).astype(jnp.bfloat16),
                         wc_ref[...],
                         preferred_element_type=jnp.float32) + wcb_ref[...]
            pf = pf.reshape(CS, B2, E) * mf_ref[...][:, :, None]
            x_s[pl.ds(c * CS, CS)] = pf.astype(jnp.bfloat16)
            pb = jnp.dot(nb_ref[...].reshape(CS * B2, E).astype(jnp.bfloat16),
                         wc_ref[...],
                         preferred_element_type=jnp.float32) + wcb_ref[...]
            pb = pb.reshape(CS, B2, E) * mb_ref[...][:, :, None]
            x_s[pl.ds((CH - 1 - c) * CS, CS)] = pb.astype(jnp.bfloat16)
            part = jnp.maximum(jnp.max(pf, axis=0), jnp.max(pb, axis=0))

            @pl.when(c == 0)
            def _():
                e_ref[...] = part

            @pl.when(c > 0)
            def _():
                e_ref[...] = jnp.maximum(e_ref[...], part)

        xf = x_s[pl.ds(c * CS, CS), :, :]
        xb = x_s[pl.ds((CH - 1 - c) * CS, CS), :, :]
        gif_s[...] = jnp.dot(xf.reshape(CS * B2, E),
                             wfih_ref[...],
                             preferred_element_type=jnp.float32) + bfih_ref[...]
        gib_s[...] = jnp.dot(xb.reshape(CS * B2, E),
                             wbih_ref[...],
                             preferred_element_type=jnp.float32) + bbih_ref[...]

        @pl.when(c == 0)
        def _():
            hf_s[...] = jnp.zeros((B2, H), jnp.float32)
            hb_s[...] = jnp.zeros((B2, H), jnp.float32)
            gmax_ref[...] = jnp.full((B2, 2 * H), -jnp.inf, jnp.float32)
            hd_s[...] = jnp.zeros((B1, 2 * HD), jnp.float32)
            dmax_ref[...] = jnp.full((B1, 2 * HD), -jnp.inf, jnp.float32)

        wfhh = wfhh_ref[...]
        bfhh = bfhh_ref[...]
        wbhh = wbhh_ref[...]
        bbhh = bbhh_ref[...]

        def step(t, carry):
            hf, hb, gmf, gmb = carry
            gif = gif_s[pl.ds(t * B2, B2), :]
            hf = _gru_step(hf, gif, wfhh, bfhh, H)
            gef_s[pl.ds(c * CS + t, 1)] = hf[None]
            gib = gib_s[pl.ds((CS - 1 - t) * B2, B2), :]
            hb = _gru_step(hb, gib, wbhh, bbhh, H)
            geb_s[pl.ds(c * CS + t, 1)] = hb[None]
            return (hf, hb, jnp.maximum(gmf, hf), jnp.maximum(gmb, hb))

        init = (hf_s[...], hb_s[...], gmax_ref[:, :H], gmax_ref[:, H:])
        hf, hb, gmf, gmb = lax.fori_loop(0, CS, step, init, unroll=24)
        hf_s[...] = hf
        hb_s[...] = hb
        gmax_ref[:, :H] = gmf
        gmax_ref[:, H:] = gmb

    @pl.when(c >= CH)
    def _():
        cc = c - CH
        af = gef_s[pl.ds(cc * CS, CS), 0:B1, :].reshape(CS * B1, H)
        bf = geb_s[pl.ds((CH - 1 - cc) * CS, CS), 0:B1, :].reshape(CS * B1, H)
        ab = gef_s[pl.ds((CH - 1 - cc) * CS, CS), 0:B1, :].reshape(CS * B1, H)
        bb = geb_s[pl.ds(cc * CS, CS), 0:B1, :].reshape(CS * B1, H)
        g1_s[...] = (
            jnp.dot(af.astype(jnp.bfloat16), w1f_ref[...],
                    preferred_element_type=jnp.float32)
            + jnp.dot(bb.astype(jnp.bfloat16), w2b_ref[...],
                      preferred_element_type=jnp.float32)
            + bih_ref[...])
        g2_s[...] = (
            jnp.dot(bf.astype(jnp.bfloat16), w2f_ref[...],
                    preferred_element_type=jnp.float32)
            + jnp.dot(ab.astype(jnp.bfloat16), w1b_ref[...],
                      preferred_element_type=jnp.float32))

        whh = whh_ref[...]
        bhh = bhh_ref[...]
        HC = 2 * HD

        def step(t, carry):
            h, dm = carry
            gi = (g1_s[pl.ds(t * B1, B1), :]
                  + g2_s[pl.ds((CS - 1 - t) * B1, B1), :])
            gh = jnp.dot(h.astype(jnp.bfloat16), whh,
                         preferred_element_type=jnp.float32) + bhh
            r = jax.nn.sigmoid(gi[:, :HC] + gh[:, :HC])
            z = jax.nn.sigmoid(gi[:, HC:2 * HC] + gh[:, HC:2 * HC])
            n = jnp.tanh(gi[:, 2 * HC:] + r * gh[:, 2 * HC:])
            h2 = (1.0 - z) * n + z * h
            return (h2, jnp.maximum(dm, h2))

        init = (hd_s[...], dmax_ref[...])
        h, dm = lax.fori_loop(0, CS, step, init, unroll=24)
        hd_s[...] = h
        dmax_ref[...] = dm

        @pl.when(c == 2 * CH - 1)
        def _():
            g = gmax_ref[...]
            d = jnp.abs(g[:B1, :] - g[B1:, :])
            y_ref[...] = jax.nn.sigmoid(
                jnp.dot(d, h2lw_ref[...], preferred_element_type=jnp.float32)
                + h2lb_ref[...])


def _encode_decode(node_tm, wc_wt, wc_b2, validf_tm,
                   wfih_t, wfhh_t, bfih, bfhh, wbih_t, wbhh_t, bbih, bbhh,
                   w1f_c, w2f_c, w1b_c, w2b_c, whh_cat, bih_cat, bhh_cat,
                   h2lw_t, h2lb):
    ewspec = [
        pl.BlockSpec((E, 3 * H), lambda c: (0, 0)),
        pl.BlockSpec((H, 3 * H), lambda c: (0, 0)),
        pl.BlockSpec((1, 3 * H), lambda c: (0, 0)),
        pl.BlockSpec((1, 3 * H), lambda c: (0, 0)),
    ]
    return pl.pallas_call(
        _ed_body,
        grid=(2 * CH,),
        in_specs=[
            pl.BlockSpec((CS, B2, E),
                         lambda c: (jnp.minimum(c, CH // 2 - 1), 0, 0)),
            pl.BlockSpec((CS, B2, E),
                         lambda c: (CH - 1 - jnp.minimum(c, CH // 2 - 1), 0, 0)),
            pl.BlockSpec((E, E), lambda c: (0, 0)),
            pl.BlockSpec((1, E), lambda c: (0, 0)),
            pl.BlockSpec((CS, B2),
                         lambda c: (jnp.minimum(c, CH // 2 - 1), 0)),
            pl.BlockSpec((CS, B2),
                         lambda c: (CH - 1 - jnp.minimum(c, CH // 2 - 1), 0)),
        ] + ewspec + ewspec + [
            pl.BlockSpec((H, 6 * HD), lambda c: (0, 0)),
            pl.BlockSpec((H, 6 * HD), lambda c: (0, 0)),
            pl.BlockSpec((H, 6 * HD), lambda c: (0, 0)),
            pl.BlockSpec((H, 6 * HD), lambda c: (0, 0)),
            pl.BlockSpec((2 * HD, 6 * HD), lambda c: (0, 0)),
            pl.BlockSpec((1, 6 * HD), lambda c: (0, 0)),
            pl.BlockSpec((1, 6 * HD), lambda c: (0, 0)),
            pl.BlockSpec((2 * H, 1), lambda c: (0, 0)),
            pl.BlockSpec((1, 1), lambda c: (0, 0)),
        ],
        out_specs=[
            pl.BlockSpec((B2, 2 * H), lambda c: (0, 0)),
            pl.BlockSpec((B2, E), lambda c: (0, 0)),
            pl.BlockSpec((B1, 2 * HD), lambda c: (0, 0)),
            pl.BlockSpec((B1, 1), lambda c: (0, 0)),
        ],
        out_shape=[
            jax.ShapeDtypeStruct((B2, 2 * H), jnp.float32),
            jax.ShapeDtypeStruct((B2, E), jnp.float32),
            jax.ShapeDtypeStruct((B1, 2 * HD), jnp.float32),
            jax.ShapeDtypeStruct((B1, 1), jnp.float32),
        ],
        scratch_shapes=[
            pltpu.VMEM((B2, H), jnp.float32),
            pltpu.VMEM((B2, H), jnp.float32),
            pltpu.VMEM((CS * B2, 3 * H), jnp.float32),
            pltpu.VMEM((CS * B2, 3 * H), jnp.float32),
            pltpu.VMEM((T, B2, H), jnp.float32),
            pltpu.VMEM((T, B2, H), jnp.float32),
            pltpu.VMEM((B1, 2 * HD), jnp.float32),
            pltpu.VMEM((CS * B1, 6 * HD), jnp.float32),
            pltpu.VMEM((CS * B1, 6 * HD), jnp.float32),
            pltpu.VMEM((T, B2, E), jnp.bfloat16),
        ],
    )(node_tm, node_tm, wc_wt, wc_b2, validf_tm, validf_tm,
      wfih_t, wfhh_t, bfih, bfhh, wbih_t, wbhh_t, bbih, bbhh,
      w1f_c, w2f_c, w1b_c, w2b_c, whh_cat, bih_cat, bhh_cat,
      h2lw_t, h2lb)


def kernel(tokens1, cu_seqlens1, tokens2, cu_seqlens2, emb, wc_w, wc_b,
           ge_wih_f, ge_whh_f, ge_bih_f, ge_bhh_f,
           ge_wih_b, ge_whh_b, ge_bih_b, ge_bhh_b,
           gd_wih_f, gd_whh_f, gd_bih_f, gd_bhh_f,
           gd_wih_b, gd_whh_b, gd_bih_b, gd_bhh_b,
           h2l_w, h2l_b):
    n1 = tokens1.shape[0]
    tokens = jnp.concatenate([tokens1, tokens2]).astype(jnp.int32)
    starts = jnp.concatenate([cu_seqlens1[:-1], cu_seqlens2[:-1] + n1])
    lens = jnp.concatenate([cu_seqlens1[1:] - cu_seqlens1[:-1],
                            cu_seqlens2[1:] - cu_seqlens2[:-1]])
    pad = T - lens
    j = jnp.arange(T, dtype=jnp.int32)
    idx = starts[:, None] + j[None, :] - pad[:, None]
    valid = j[None, :] >= pad[:, None]
    idxc = jnp.clip(idx, 0, tokens.shape[0] - 1).astype(jnp.int32)
    tok_padded = jnp.take(tokens, idxc.T.reshape(-1), axis=0)
    node = _sc_gather(emb, tok_padded)

    validf_tm = valid.T.astype(jnp.float32)
    bf16 = jnp.bfloat16

    def _ilv(wf, wb):
        parts = []
        for g in range(3):
            parts.append(wf[:, g * HD:(g + 1) * HD])
            parts.append(wb[:, g * HD:(g + 1) * HD])
        return jnp.concatenate(parts, axis=1)

    zih = jnp.zeros((H, 3 * HD), jnp.float32)
    zhh = jnp.zeros((HD, 3 * HD), jnp.float32)
    whh_cat = jnp.concatenate([
        _ilv(gd_whh_f.T, zhh), _ilv(zhh, gd_whh_b.T)], axis=0).astype(bf16)
    bih_cat = _ilv(gd_bih_f.reshape(1, -1), gd_bih_b.reshape(1, -1))
    bhh_cat = _ilv(gd_bhh_f.reshape(1, -1), gd_bhh_b.reshape(1, -1))

    gmax, e, dmax, y = _encode_decode(
        node.reshape(T, B2, E), wc_w.T.astype(bf16), wc_b.reshape(1, -1),
        validf_tm,
        ge_wih_f.T.astype(bf16), ge_whh_f.T.astype(bf16),
        ge_bih_f.reshape(1, -1), ge_bhh_f.reshape(1, -1),
        ge_wih_b.T.astype(bf16), ge_whh_b.T.astype(bf16),
        ge_bih_b.reshape(1, -1), ge_bhh_b.reshape(1, -1),
        _ilv(gd_wih_f[:, :H].T, zih).astype(bf16),
        _ilv(gd_wih_f[:, H:].T, zih).astype(bf16),
        _ilv(zih, gd_wih_b[:, :H].T).astype(bf16),
        _ilv(zih, gd_wih_b[:, H:].T).astype(bf16),
        whh_cat, bih_cat, bhh_cat,
        h2l_w.T, h2l_b.reshape(1, 1))

    return (e[:B1], gmax[:B1], dmax, y)
